# baseline (speedup 1.0000x reference)
import functools

import jax
import jax.numpy as jnp
from jax import lax
from jax.experimental import pallas as pl
from jax.experimental.pallas import tpu as pltpu
from jax.experimental.pallas import tpu_sc as plsc

HID = 768
EPS = 1e-12
T = 2048


def _make_sc_gather(n_tokens: int):
    info = plsc.get_sparse_core_info()
    nc, ns = info.num_cores, info.num_subcores
    nw = nc * ns
    b_per_w = n_tokens // nw
    chunk = 32
    n_chunks = b_per_w // chunk
    mesh = plsc.VectorSubcoreMesh(core_axis_name="c", subcore_axis_name="s")

    @functools.partial(
        pl.kernel,
        mesh=mesh,
        out_type=jax.ShapeDtypeStruct((n_tokens, HID), jnp.float32),
        scratch_types=[
            pltpu.VMEM((b_per_w,), jnp.int32),
            pltpu.VMEM((chunk, HID), jnp.float32),
            pltpu.VMEM((chunk, HID), jnp.float32),
            pltpu.SemaphoreType.DMA,
            pltpu.SemaphoreType.DMA,
            pltpu.SemaphoreType.DMA,
            pltpu.SemaphoreType.DMA,
        ],
    )
    def gather_k(idx_hbm, table_hbm, out_hbm, idx_v, rows0, rows1,
                 si0, si1, so0, so1):
        wid = lax.axis_index("s") * nc + lax.axis_index("c")
        base = wid * b_per_w
        pltpu.sync_copy(idx_hbm.at[pl.ds(base, b_per_w)], idx_v)
        bufs = (rows0, rows1)
        sin = (si0, si1)
        sout = (so0, so1)
        ins = [None] * n_chunks
        outs = [None] * n_chunks
        ins[0] = pltpu.async_copy(
            table_hbm.at[idx_v.at[pl.ds(0, chunk)]], rows0, si0)
        for c in range(n_chunks):
            if c + 1 < n_chunks:
                if c >= 1:
                    outs[c - 1].wait()
                ins[c + 1] = pltpu.async_copy(
                    table_hbm.at[idx_v.at[pl.ds((c + 1) * chunk, chunk)]],
                    bufs[(c + 1) % 2], sin[(c + 1) % 2])
            ins[c].wait()
            outs[c] = pltpu.async_copy(
                bufs[c % 2], out_hbm.at[pl.ds(base + c * chunk, chunk)],
                sout[c % 2])
        if n_chunks >= 2:
            outs[n_chunks - 2].wait()
        outs[n_chunks - 1].wait()

    return gather_k


def _ln_body(w_ref, p_ref, ttf_ref, te_ref, g_ref, b_ref, o_ref):
    w = w_ref[...]
    p = p_ref[...]
    ttf = ttf_ref[0, 0, :][:, None]
    t0 = te_ref[0, :][None, :]
    t1 = te_ref[1, :][None, :]
    e = w + p + t0 + ttf * (t1 - t0)
    mu = jnp.mean(e, axis=1, keepdims=True)
    ex2 = jnp.mean(e * e, axis=1, keepdims=True)
    var = ex2 - mu * mu
    o_ref[0] = (e - mu) * lax.rsqrt(var + EPS) * g_ref[...] + b_ref[...]


def kernel(input_ids, token_type_ids, word_emb, pos_emb, type_emb, ln_gamma,
           ln_beta):
    B, S = input_ids.shape
    n = B * S
    spt = S // T

    ids = input_ids.astype(jnp.int32).reshape(n)
    words = _make_sc_gather(n)(ids, word_emb)

    ttf = token_type_ids.astype(jnp.float32).reshape(n // T, 1, T)

    out = pl.pallas_call(
        _ln_body,
        grid=(spt, B),
        in_specs=[
            pl.BlockSpec((T, HID), lambda s, b: (b * spt + s, 0)),
            pl.BlockSpec((T, HID), lambda s, b: (s, 0)),
            pl.BlockSpec((1, 1, T), lambda s, b: (b * spt + s, 0, 0)),
            pl.BlockSpec((2, HID), lambda s, b: (0, 0)),
            pl.BlockSpec((1, HID), lambda s, b: (0, 0)),
            pl.BlockSpec((1, HID), lambda s, b: (0, 0)),
        ],
        out_specs=pl.BlockSpec((1, T, HID), lambda s, b: (b, s, 0)),
        out_shape=jax.ShapeDtypeStruct((B, S, HID), jnp.float32),
    )(words, pos_emb, ttf, type_emb,
      ln_gamma.reshape(1, HID), ln_beta.reshape(1, HID))
    return out


# speedup vs baseline: 1.0057x; 1.0057x over previous
import functools

import jax
import jax.numpy as jnp
from jax import lax
from jax.experimental import pallas as pl
from jax.experimental.pallas import tpu as pltpu
from jax.experimental.pallas import tpu_sc as plsc

HID = 768
EPS = 1e-12
T = 2048


def _make_sc_gather(n_tokens: int):
    info = plsc.get_sparse_core_info()
    nc, ns = info.num_cores, info.num_subcores
    nw = nc * ns
    b_per_w = n_tokens // nw
    chunk = 64
    n_chunks = b_per_w // chunk
    mesh = plsc.VectorSubcoreMesh(core_axis_name="c", subcore_axis_name="s")

    @functools.partial(
        pl.kernel,
        mesh=mesh,
        out_type=jax.ShapeDtypeStruct((n_tokens, HID), jnp.float32),
        scratch_types=[
            pltpu.VMEM((b_per_w,), jnp.int32),
            pltpu.VMEM((chunk, HID), jnp.float32),
            pltpu.VMEM((chunk, HID), jnp.float32),
            pltpu.SemaphoreType.DMA,
            pltpu.SemaphoreType.DMA,
            pltpu.SemaphoreType.DMA,
            pltpu.SemaphoreType.DMA,
        ],
    )
    def gather_k(idx_hbm, table_hbm, out_hbm, idx_v, rows0, rows1,
                 si0, si1, so0, so1):
        wid = lax.axis_index("s") * nc + lax.axis_index("c")
        base = wid * b_per_w
        pltpu.sync_copy(idx_hbm.at[pl.ds(base, b_per_w)], idx_v)
        bufs = (rows0, rows1)
        sin = (si0, si1)
        sout = (so0, so1)
        ins = [None] * n_chunks
        outs = [None] * n_chunks
        ins[0] = pltpu.async_copy(
            table_hbm.at[idx_v.at[pl.ds(0, chunk)]], rows0, si0)
        for c in range(n_chunks):
            if c + 1 < n_chunks:
                if c >= 1:
                    outs[c - 1].wait()
                ins[c + 1] = pltpu.async_copy(
                    table_hbm.at[idx_v.at[pl.ds((c + 1) * chunk, chunk)]],
                    bufs[(c + 1) % 2], sin[(c + 1) % 2])
            ins[c].wait()
            outs[c] = pltpu.async_copy(
                bufs[c % 2], out_hbm.at[pl.ds(base + c * chunk, chunk)],
                sout[c % 2])
        if n_chunks >= 2:
            outs[n_chunks - 2].wait()
        outs[n_chunks - 1].wait()

    return gather_k


def _ln_body(w_ref, p_ref, ttf_ref, te_ref, g_ref, b_ref, o_ref):
    w = w_ref[...]
    p = p_ref[...]
    ttf = ttf_ref[0, 0, :][:, None]
    t0 = te_ref[0, :][None, :]
    t1 = te_ref[1, :][None, :]
    e = w + p + t0 + ttf * (t1 - t0)
    mu = jnp.mean(e, axis=1, keepdims=True)
    ex2 = jnp.mean(e * e, axis=1, keepdims=True)
    var = ex2 - mu * mu
    o_ref[0] = (e - mu) * lax.rsqrt(var + EPS) * g_ref[...] + b_ref[...]


def kernel(input_ids, token_type_ids, word_emb, pos_emb, type_emb, ln_gamma,
           ln_beta):
    B, S = input_ids.shape
    n = B * S
    spt = S // T

    ids = input_ids.astype(jnp.int32).reshape(n)
    words = _make_sc_gather(n)(ids, word_emb)

    ttf = token_type_ids.astype(jnp.float32).reshape(n // T, 1, T)

    out = pl.pallas_call(
        _ln_body,
        grid=(spt, B),
        in_specs=[
            pl.BlockSpec((T, HID), lambda s, b: (b * spt + s, 0)),
            pl.BlockSpec((T, HID), lambda s, b: (s, 0)),
            pl.BlockSpec((1, 1, T), lambda s, b: (b * spt + s, 0, 0)),
            pl.BlockSpec((2, HID), lambda s, b: (0, 0)),
            pl.BlockSpec((1, HID), lambda s, b: (0, 0)),
            pl.BlockSpec((1, HID), lambda s, b: (0, 0)),
        ],
        out_specs=pl.BlockSpec((1, T, HID), lambda s, b: (b, s, 0)),
        out_shape=jax.ShapeDtypeStruct((B, S, HID), jnp.float32),
    )(words, pos_emb, ttf, type_emb,
      ln_gamma.reshape(1, HID), ln_beta.reshape(1, HID))
    return out
